# baseline (device time: 101878 ns/iter reference)
import jax
import jax.numpy as jnp
from jax import lax
from jax.experimental import pallas as pl
from jax.experimental.pallas import tpu as pltpu

N_Z = 4
VOCAB_PER_SHARD = 8192
BLK = 64
MAXB = 20


def _gather_allgather(meta, perm, grow, srcrow, E):
    t = perm.shape[0]
    v, d = E.shape
    g_rows = t + N_Z * BLK

    def body(meta_ref, perm_ref, grow_ref, srcrow_ref, e_ref, out_ref,
             g_ref, rx1, tx1, rx2, tx2, rx3, tx3):
        my_x = lax.axis_index("x")
        my_y = lax.axis_index("y")
        my_z = lax.axis_index("z")

        def boff(z):
            return meta_ref[z]

        def nblk(z):
            return meta_ref[N_Z + z]

        def first(z):
            return meta_ref[2 * N_Z + z]

        def count(z):
            return meta_ref[3 * N_Z + z]

        def loop(n, fn):
            lax.fori_loop(0, n, lambda k, c: (fn(k), c)[1], 0)

        def rcopy(row, ssem, rsem, zdst):
            row = pl.multiple_of(row, BLK)
            return pltpu.make_async_remote_copy(
                src_ref=g_ref.at[pl.ds(row, BLK), :],
                dst_ref=g_ref.at[pl.ds(row, BLK), :],
                send_sem=ssem, recv_sem=rsem,
                device_id=(my_x, my_y, zdst),
                device_id_type=pl.DeviceIdType.MESH,
            )

        my_boff = boff(my_z)
        my_first = first(my_z)

        def gat(j, carry):
            src = srcrow_ref[my_first + j]
            g_ref[pl.ds(my_boff + j, 1), :] = e_ref[pl.ds(src, 1), :]
            return carry

        lax.fori_loop(0, count(my_z), gat, 0)

        is_edge = jnp.logical_or(my_z == 0, my_z == N_Z - 1)
        barrier_sem = pltpu.get_barrier_semaphore()

        @pl.when(is_edge)
        def _():
            mate = jnp.where(my_z == 0, 1, N_Z - 2)
            pl.semaphore_signal(
                barrier_sem, inc=1, device_id=(my_x, my_y, mate),
                device_id_type=pl.DeviceIdType.MESH,
            )
            pl.semaphore_wait(barrier_sem, 1)

        @pl.when(jnp.logical_not(is_edge))
        def _():
            for nbr in (my_z - 1, my_z + 1):
                pl.semaphore_signal(
                    barrier_sem, inc=1, device_id=(my_x, my_y, nbr),
                    device_id_type=pl.DeviceIdType.MESH,
                )
            pl.semaphore_wait(barrier_sem, 2)

        nblk_sum = nblk(0) + nblk(1) + nblk(2) + nblk(3)

        @pl.when(is_edge)
        def _():
            own = my_z
            mate = jnp.where(my_z == 0, 1, N_Z - 2)
            nb_own = nblk(own)
            loop(nb_own, lambda b: rcopy(
                boff(own) + BLK * b, tx1.at[b], rx1.at[b], mate).start())
            loop(nblk_sum - nb_own, lambda j: rcopy(
                0, tx3.at[j], rx3.at[j], mate).wait_recv())
            loop(nb_own, lambda b: rcopy(
                boff(own) + BLK * b, tx1.at[b], rx1.at[b], mate).wait_send())

        @pl.when(jnp.logical_not(is_edge))
        def _():
            is_lo = my_z == 1
            own = jnp.where(is_lo, 1, N_Z - 2)
            eset = jnp.where(is_lo, 0, N_Z - 1)
            o_a = jnp.where(is_lo, N_Z - 2, 1)
            o_b = jnp.where(is_lo, N_Z - 1, 0)
            nb_own, nb_e = nblk(own), nblk(eset)
            nb_a, nb_b = nblk(o_a), nblk(o_b)

            loop(nb_own, lambda b: rcopy(
                boff(own) + BLK * b, tx3.at[b], rx3.at[b], eset).start())
            loop(nb_own, lambda b: rcopy(
                boff(own) + BLK * b, tx2.at[b], rx2.at[b], o_a).start())

            def f1(b):
                rcopy(boff(eset) + BLK * b,
                      tx1.at[b], rx1.at[b], eset).wait_recv()
                rcopy(boff(eset) + BLK * b,
                      tx2.at[nb_own + b], rx2.at[nb_own + b], o_a).start()

            loop(nb_e, f1)

            def f2(j):
                row = jnp.where(j < nb_a,
                                boff(o_a) + BLK * j,
                                boff(o_b) + BLK * (j - nb_a))
                rcopy(row, tx2.at[0], rx2.at[j], o_a).wait_recv()
                rcopy(row, tx3.at[nb_own + j],
                      rx3.at[nb_own + j], eset).start()

            loop(nb_a + nb_b, f2)

            loop(nb_own, lambda b: rcopy(
                boff(own) + BLK * b, tx3.at[b], rx3.at[b], eset).wait_send())
            loop(nb_a + nb_b, lambda j: rcopy(
                0, tx3.at[nb_own + j], rx3.at[nb_own + j], eset).wait_send())
            loop(nb_own, lambda b: rcopy(
                boff(own) + BLK * b, tx2.at[b], rx2.at[b], o_a).wait_send())
            loop(nb_e, lambda b: rcopy(
                0, tx2.at[nb_own + b], rx2.at[nb_own + b], o_a).wait_send())

        def sc(i, carry):
            out_ref[pl.ds(perm_ref[i], 1), :] = g_ref[pl.ds(grow_ref[i], 1), :]
            return carry

        lax.fori_loop(0, t, sc, 0)

    smem = pl.BlockSpec(memory_space=pltpu.SMEM)
    sem = pltpu.SemaphoreType.DMA((MAXB,))
    return pl.pallas_call(
        body,
        out_shape=jax.ShapeDtypeStruct((t, d), jnp.float32),
        in_specs=[smem, smem, smem, smem,
                  pl.BlockSpec(memory_space=pltpu.VMEM)],
        out_specs=pl.BlockSpec(memory_space=pltpu.VMEM),
        scratch_shapes=[
            pltpu.VMEM((g_rows, d), jnp.float32),
            sem, sem, sem, sem, sem, sem,
        ],
        compiler_params=pltpu.CompilerParams(
            collective_id=0, vmem_limit_bytes=64 * 1024 * 1024
        ),
    )(meta, perm, grow, srcrow, E)


def kernel(ids, E):
    t = ids.shape[0]
    owner = (ids // VOCAB_PER_SHARD).astype(jnp.int32)
    perm = jnp.argsort(owner).astype(jnp.int32)
    counts = jnp.bincount(owner, length=N_Z).astype(jnp.int32)
    nblk = (counts + BLK - 1) // BLK
    boff = (BLK * (jnp.cumsum(nblk) - nblk)).astype(jnp.int32)
    first = (jnp.cumsum(counts) - counts).astype(jnp.int32)
    owner_s = owner[perm]
    srcrow = (ids[perm] % VOCAB_PER_SHARD).astype(jnp.int32)
    grow = (boff[owner_s] + (jnp.arange(t, dtype=jnp.int32)
                             - first[owner_s])).astype(jnp.int32)
    meta = jnp.concatenate([boff, nblk, first, counts]).astype(jnp.int32)
    return _gather_allgather(meta, perm, grow, srcrow, E)
